# Initial kernel scaffold; baseline (speedup 1.0000x reference)
import jax
import jax.numpy as jnp
from jax.experimental import pallas as pl


def kernel(x, edge_index, edge_attr, Wq, bq, Wk, bk, Wv, bv, Wr, br, Whi, Whj, ln1_g, ln1_b, W1, b1, W2, b2, ln2_g, ln2_b, Wl, bl, Wl2, bl2):
    raise NotImplementedError("write your pallas kernel here")


# speedup vs baseline: 3.0235x; 3.0235x over previous
import functools

import jax
import jax.numpy as jnp
from jax import lax
from jax.experimental import pallas as pl
from jax.experimental.pallas import tpu as pltpu
from jax.experimental.pallas import tpu_sc as plsc

N = 10000
E = 320000
D = 128
NC = 2
NS = 16
NW = NC * NS
EPW = E // NW
C = 80
NCHUNK = EPW // C
RPT = 624
REM = N - RPT * NS

_SC_MESH = plsc.VectorSubcoreMesh(core_axis_name="c", subcore_axis_name="s")


def _shuffle(v, sh):
    idx = (jnp.arange(16, dtype=jnp.int32) ^ sh)[:, None]
    return lax.gather(
        v, idx,
        dimension_numbers=lax.GatherDimensionNumbers(
            offset_dims=(), collapsed_slice_dims=(0,), start_index_map=(0,)),
        slice_sizes=(1,),
        mode=lax.GatherScatterMode.PROMISE_IN_BOUNDS)


def _lane_sum(v):
    for sh in (1, 2, 4, 8):
        v = v + _shuffle(v, sh)
    return v


def _lane_max(v):
    for sh in (1, 2, 4, 8):
        v = jnp.maximum(v, _shuffle(v, sh))
    return v


_BLK = 1000
_GRID = N // _BLK


def _proj_body(x_ref, w_ref, b_ref, q_ref, k_ref, v_ref, hi_ref, hj_ref,
               root_ref):
    y = jnp.dot(x_ref[...], w_ref[...], preferred_element_type=jnp.float32)
    y = y + b_ref[...]
    q_ref[...] = y[:, 0:128]
    k_ref[...] = y[:, 128:256]
    v_ref[...] = y[:, 256:384]
    hi_ref[...] = y[:, 384:512]
    hj_ref[...] = y[:, 512:640]
    root_ref[...] = y[:, 640:768]


def _proj(x, w_all, b_all):
    outs = [jax.ShapeDtypeStruct((N, D), jnp.float32) for _ in range(6)]
    return pl.pallas_call(
        _proj_body,
        grid=(_GRID,),
        in_specs=[
            pl.BlockSpec((_BLK, D), lambda i: (i, 0)),
            pl.BlockSpec((D, 768), lambda i: (0, 0)),
            pl.BlockSpec((1, 768), lambda i: (0, 0)),
        ],
        out_specs=[pl.BlockSpec((_BLK, D), lambda i: (i, 0))] * 6,
        out_shape=outs,
    )(x, w_all, b_all)


def _wfuse_body(wl_ref, wl2_ref, bl_ref, bl2_ref, wf_ref, bf_ref):
    wf_ref[...] = jnp.dot(wl_ref[...], wl2_ref[...],
                          preferred_element_type=jnp.float32)
    bf_ref[...] = jnp.dot(bl_ref[...], wl2_ref[...],
                          preferred_element_type=jnp.float32) + bl2_ref[...]


def _wfuse(Wl, Wl2, bl, bl2):
    return pl.pallas_call(
        _wfuse_body,
        out_shape=[jax.ShapeDtypeStruct((D, D), jnp.float32),
                   jax.ShapeDtypeStruct((1, D), jnp.float32)],
    )(Wl, Wl2, bl, bl2)


@functools.partial(
    pl.kernel,
    out_type=[
        jax.ShapeDtypeStruct((E,), jnp.float32),
        jax.ShapeDtypeStruct((NW * 16,), jnp.float32),
        jax.ShapeDtypeStruct((NW * 16,), jnp.float32),
    ],
    mesh=_SC_MESH,
    scratch_types=[
        pltpu.VMEM((C,), jnp.int32),
        pltpu.VMEM((C,), jnp.int32),
        pltpu.VMEM((C, D), jnp.float32),
        pltpu.VMEM((C, D), jnp.float32),
        pltpu.VMEM((C,), jnp.float32),
        pltpu.VMEM((16,), jnp.float32),
        pltpu.SemaphoreType.DMA,
        pltpu.SemaphoreType.DMA,
    ],
)
def _s1(src_hbm, dst_hbm, q_hbm, k_hbm, att_out, m_out, s_out,
        sidx, didx, qrows, krows, attb, statb, sem1, sem2):
    cid = lax.axis_index("c")
    sid = lax.axis_index("s")
    wid = sid * NC + cid
    base_w = wid * EPW

    def chunk(i, carry):
        m_run, s_run = carry
        base = base_w + i * C
        pltpu.sync_copy(src_hbm.at[pl.ds(base, C)], sidx)
        pltpu.sync_copy(dst_hbm.at[pl.ds(base, C)], didx)
        cp_q = pltpu.async_copy(q_hbm.at[didx], qrows, sem1)
        cp_k = pltpu.async_copy(k_hbm.at[sidx], krows, sem2)
        cp_q.wait()
        cp_k.wait()

        lane = jnp.arange(16, dtype=jnp.int32)

        def grp(t, carry2):
            m_r, s_r = carry2
            vec = jnp.zeros((16,), jnp.float32)
            for j2 in range(16):
                j = t * 16 + j2
                acc = qrows[j, 0:16] * krows[j, 0:16]
                for r in range(1, 8):
                    acc = acc + (qrows[j, 16 * r:16 * r + 16]
                                 * krows[j, 16 * r:16 * r + 16])
                vec = jnp.where(lane == j2, _lane_sum(acc), vec)
            attb[pl.ds(t * 16, 16)] = vec
            m_n = jnp.maximum(m_r, vec)
            s_n = s_r * jnp.exp(m_r - m_n) + jnp.exp(vec - m_n)
            return (m_n, s_n)

        m_run, s_run = lax.fori_loop(0, C // 16, grp, (m_run, s_run))
        pltpu.sync_copy(attb, att_out.at[pl.ds(base, C)])
        return (m_run, s_run)

    m0 = jnp.full((16,), -1e30, dtype=jnp.float32)
    s0 = jnp.zeros((16,), dtype=jnp.float32)
    m_run, s_run = lax.fori_loop(0, NCHUNK, chunk, (m0, s0))

    m_fin = _lane_max(m_run)
    s_fin = _lane_sum(s_run * jnp.exp(m_run - m_fin))
    statb[...] = m_fin
    pltpu.sync_copy(statb, m_out.at[pl.ds(wid * 16, 16)])
    statb[...] = s_fin
    pltpu.sync_copy(statb, s_out.at[pl.ds(wid * 16, 16)])


@functools.partial(
    pl.kernel,
    out_type=jax.ShapeDtypeStruct((NC, N, D), jnp.float32),
    mesh=_SC_MESH,
    scratch_types=[
        pltpu.VMEM((C,), jnp.int32),
        pltpu.VMEM((C,), jnp.int32),
        pltpu.VMEM((C, D), jnp.float32),
        pltpu.VMEM((C, D), jnp.float32),
        pltpu.VMEM((C, D), jnp.float32),
        pltpu.VMEM((C,), jnp.float32),
        pltpu.VMEM((C,), jnp.float32),
        pltpu.VMEM((C, D), jnp.float32),
        pltpu.VMEM((NW * 16,), jnp.float32),
        pltpu.VMEM((NW * 16,), jnp.float32),
        pltpu.SemaphoreType.DMA,
        pltpu.SemaphoreType.DMA,
        pltpu.SemaphoreType.DMA,
        pltpu.VMEM_SHARED((N, D), jnp.float32),
    ],
)
def _s2(src_hbm, dst_hbm, v_hbm, hi_hbm, hj_hbm, ea_hbm, att_hbm,
        m_hbm, s_hbm, zseg_hbm, aggr_out,
        sidx, didx, vrows, hirows, hjrows, attb, pbuf, msgb,
        mtab, stab, sem1, sem2, sem3, aggr):
    cid = lax.axis_index("c")
    sid = lax.axis_index("s")
    wid = sid * NC + cid
    base_w = wid * EPW

    pltpu.sync_copy(zseg_hbm, aggr.at[pl.ds(sid * RPT, RPT)])

    @pl.when(sid == NS - 1)
    def _zero_tail():
        pltpu.sync_copy(zseg_hbm.at[pl.ds(0, REM)],
                        aggr.at[pl.ds(NS * RPT, REM)])

    pltpu.sync_copy(m_hbm, mtab)
    pltpu.sync_copy(s_hbm, stab)

    def mred(w, m_r):
        return jnp.maximum(m_r, mtab[pl.ds(w * 16, 16)])

    m_glob = lax.fori_loop(0, NW, mred, jnp.full((16,), -1e30, jnp.float32))

    def sred(w, s_r):
        return s_r + stab[pl.ds(w * 16, 16)] * jnp.exp(mtab[pl.ds(w * 16, 16)] - m_glob)

    s_glob = lax.fori_loop(0, NW, sred, jnp.zeros((16,), jnp.float32))
    inv_s = 1.0 / s_glob

    plsc.subcore_barrier()

    def chunk(i, _):
        base = base_w + i * C
        pltpu.sync_copy(src_hbm.at[pl.ds(base, C)], sidx)
        pltpu.sync_copy(dst_hbm.at[pl.ds(base, C)], didx)
        cp_v = pltpu.async_copy(v_hbm.at[sidx], vrows, sem1)
        cp_hi = pltpu.async_copy(hi_hbm.at[sidx], hirows, sem2)
        cp_hj = pltpu.async_copy(hj_hbm.at[didx], hjrows, sem3)
        pltpu.sync_copy(ea_hbm.at[pl.ds(base, C)], msgb)
        pltpu.sync_copy(att_hbm.at[pl.ds(base, C)], attb)

        def pvec(t, _):
            a = attb[pl.ds(t * 16, 16)]
            pbuf[pl.ds(t * 16, 16)] = jnp.exp(a - m_glob) * inv_s
            return 0

        lax.fori_loop(0, C // 16, pvec, 0)
        cp_v.wait()
        cp_hi.wait()
        cp_hj.wait()

        def grp(t, _):
            pv = pbuf[pl.ds(t * 16, 16)]
            for j2 in range(16):
                j = t * 16 + j2
                p = pv[j2]
                for r in range(8):
                    sl = pl.ds(16 * r, 16)
                    z = msgb[j, sl] + hirows[j, sl] + hjrows[j, sl]
                    gate = 1.0 / (1.0 + jnp.exp(-z))
                    msgb[j, sl] = p * vrows[j, sl] * gate
            return 0

        lax.fori_loop(0, C // 16, grp, 0)
        pltpu.sync_copy(msgb, aggr.at[didx], add=True)
        return 0

    lax.fori_loop(0, NCHUNK, chunk, 0)
    plsc.subcore_barrier()
    pltpu.sync_copy(aggr.at[pl.ds(sid * RPT, RPT)],
                    aggr_out.at[cid, pl.ds(sid * RPT, RPT)])

    @pl.when(sid == NS - 1)
    def _export_tail():
        pltpu.sync_copy(aggr.at[pl.ds(NS * RPT, REM)],
                        aggr_out.at[cid, pl.ds(NS * RPT, REM)])


def _ln(y, g, b):
    m = jnp.mean(y, axis=-1, keepdims=True)
    var = jnp.mean((y - m) ** 2, axis=-1, keepdims=True)
    return (y - m) / jnp.sqrt(var + 1e-5) * g + b


def _tail_body(ag_ref, root_ref, g1_ref, b1g_ref, W1_ref, b1_ref, W2_ref,
               b2_ref, g2_ref, b2g_ref, wf_ref, bf_ref, out_ref):
    a = ag_ref[0] + ag_ref[1] + root_ref[...]
    ss = _ln(a, g1_ref[...], b1g_ref[...])
    h = jnp.maximum(
        jnp.dot(ss, W1_ref[...], preferred_element_type=jnp.float32)
        + b1_ref[...], 0.0)
    h2 = jnp.dot(h, W2_ref[...], preferred_element_type=jnp.float32) + b2_ref[...]
    o = _ln(a + h2, g2_ref[...], b2g_ref[...])
    y = jnp.dot(o, wf_ref[...], preferred_element_type=jnp.float32) + bf_ref[...]
    out_ref[...] = jnp.where(y >= 0, y, 0.01 * y)


def _tail(aggr2, root, ln1_g, ln1_b, W1, b1, W2, b2, ln2_g, ln2_b, wf, bf):
    return pl.pallas_call(
        _tail_body,
        grid=(_GRID,),
        in_specs=[
            pl.BlockSpec((NC, _BLK, D), lambda i: (0, i, 0)),
            pl.BlockSpec((_BLK, D), lambda i: (i, 0)),
            pl.BlockSpec((1, D), lambda i: (0, 0)),
            pl.BlockSpec((1, D), lambda i: (0, 0)),
            pl.BlockSpec((D, 512), lambda i: (0, 0)),
            pl.BlockSpec((1, 512), lambda i: (0, 0)),
            pl.BlockSpec((512, D), lambda i: (0, 0)),
            pl.BlockSpec((1, D), lambda i: (0, 0)),
            pl.BlockSpec((1, D), lambda i: (0, 0)),
            pl.BlockSpec((1, D), lambda i: (0, 0)),
            pl.BlockSpec((D, D), lambda i: (0, 0)),
            pl.BlockSpec((1, D), lambda i: (0, 0)),
        ],
        out_specs=pl.BlockSpec((_BLK, D), lambda i: (i, 0)),
        out_shape=jax.ShapeDtypeStruct((N, D), jnp.float32),
    )(aggr2, root, ln1_g, ln1_b, W1, b1, W2, b2, ln2_g, ln2_b, wf, bf)


def kernel(x, edge_index, edge_attr, Wq, bq, Wk, bk, Wv, bv, Wr, br, Whi, Whj,
           ln1_g, ln1_b, W1, b1, W2, b2, ln2_g, ln2_b, Wl, bl, Wl2, bl2):
    w_all = jnp.concatenate([Wq, Wk, Wv, Whi, Whj, Wr], axis=1)
    zb = jnp.zeros_like(bq)
    b_all = jnp.concatenate([bq, bk, bv, zb, zb, br])[None, :]
    q, k, v, hi, hj, root = _proj(x, w_all, b_all)
    wf, bf = _wfuse(Wl, Wl2, bl[None, :], bl2[None, :])

    src = edge_index[0]
    dst = edge_index[1]
    att, m_w, s_w = _s1(src, dst, q, k)
    zseg = jnp.zeros((RPT, D), jnp.float32)
    aggr2 = _s2(src, dst, v, hi, hj, edge_attr, att, m_w, s_w, zseg)

    return _tail(aggr2, root, ln1_g[None, :], ln1_b[None, :], W1, b1[None, :],
                 W2, b2[None, :], ln2_g[None, :], ln2_b[None, :], wf, bf)
